# baseline (speedup 1.0000x reference)
import functools

import jax
import jax.numpy as jnp
from jax import lax
from jax.experimental import pallas as pl
from jax.experimental.pallas import tpu as pltpu
from jax.experimental.pallas import tpu_sc as plsc

N_TYPES = 100
D = 512
B = 100000
NC = 2
NS = 16
NW = NC * NS
C = 80
NSLOT = 40
RPW = NSLOT * C
LAST_N = B - (NW - 1) * RPW


def _emb_body(idx_hbm, w_hbm, out_hbm, idx_v, rows0, rows1, g0, g1, o0, o1):
    wid = lax.axis_index("s") * NC + lax.axis_index("c")
    base = wid * RPW
    nval = jnp.where(wid == NW - 1, LAST_N // C, NSLOT)

    @pl.when(wid == NW - 1)
    def _():
        pltpu.sync_copy(idx_hbm.at[pl.ds(base, LAST_N)], idx_v.at[pl.ds(0, LAST_N)])

    @pl.when(wid != NW - 1)
    def _():
        pltpu.sync_copy(idx_hbm.at[pl.ds(base, RPW)], idx_v)

    def gather_start(j, rows, sem):
        pltpu.async_copy(w_hbm.at[idx_v.at[pl.ds(j * C, C)]], rows, sem)

    def gather_wait(j, rows, sem):
        pltpu.make_async_copy(w_hbm.at[idx_v.at[pl.ds(j * C, C)]], rows, sem).wait()

    def scatter_start(j, rows, sem):
        pltpu.async_copy(rows, out_hbm.at[pl.ds(base + j * C, C)], sem)

    def scatter_wait(j, rows, sem):
        pltpu.make_async_copy(rows, out_hbm.at[pl.ds(base + j * C, C)], sem).wait()

    gather_start(0, rows0, g0)
    gather_start(1, rows1, g1)

    def step(t, carry):
        j0 = 2 * t
        j1 = j0 + 1
        gather_wait(j0, rows0, g0)
        scatter_start(j0, rows0, o0)
        gather_wait(j1, rows1, g1)
        scatter_start(j1, rows1, o1)

        @pl.when(j0 + 2 < nval)
        def _():
            scatter_wait(j0, rows0, o0)
            gather_start(j0 + 2, rows0, g0)

        @pl.when(j1 + 2 < nval)
        def _():
            scatter_wait(j1, rows1, o1)
            gather_start(j1 + 2, rows1, g1)

        return carry

    lax.fori_loop(0, nval // 2, step, 0)
    scatter_wait(0, rows0, o0)
    scatter_wait(0, rows1, o1)


@jax.jit
def _emb(idx, w):
    mesh = plsc.VectorSubcoreMesh(core_axis_name="c", subcore_axis_name="s")
    f = functools.partial(
        pl.kernel,
        mesh=mesh,
        out_type=jax.ShapeDtypeStruct((B, D), jnp.float32),
        scratch_types=[
            pltpu.VMEM((RPW,), jnp.int32),
            pltpu.VMEM((C, D), jnp.float32),
            pltpu.VMEM((C, D), jnp.float32),
            pltpu.SemaphoreType.DMA,
            pltpu.SemaphoreType.DMA,
            pltpu.SemaphoreType.DMA,
            pltpu.SemaphoreType.DMA,
        ],
    )(_emb_body)
    return f(idx, w)


def kernel(atom_numbers, W):
    idx = jnp.squeeze(atom_numbers, axis=-1)
    return _emb(idx, W)
